# baseline (speedup 1.0000x reference)
import functools
import math

import jax
import jax.numpy as jnp
from jax import lax
from jax.experimental import pallas as pl
from jax.experimental.pallas import tpu as pltpu
from jax.experimental.pallas import tpu_sc as plsc

MAX_T = 4
D = 1280
B = 8
N = 1025
BT = B * MAX_T


def _vgather(vec, idx):
    return lax.gather(
        vec, idx[:, None],
        dimension_numbers=lax.GatherDimensionNumbers(
            offset_dims=(), collapsed_slice_dims=(0,), start_index_map=(0,)),
        slice_sizes=(1,),
        mode=lax.GatherScatterMode.PROMISE_IN_BOUNDS)


def _sc_gather_body(ar_hbm, emb_hbm, pe_hbm, ar_v, idx_v, rows_v, sem):
    cid = lax.axis_index("c")
    sid = lax.axis_index("s")

    @pl.when(jnp.logical_and(cid == 0, sid == 0))
    def _():
        pltpu.sync_copy(ar_hbm, ar_v)
        ar = ar_v[...]
        for j in range(2):
            lane = lax.broadcasted_iota(jnp.int32, (16,), 0)
            wid = lane + j * 16
            b = lax.div(wid, 4)
            t = wid - b * 4
            h = _vgather(ar, 2 * b)
            w = _vgather(ar, 2 * b + 1)
            ws = jnp.maximum(w, 1)
            r = lax.div(t, ws)
            c = t - r * ws
            valid = t < h * w
            idx = jnp.where(valid, r * MAX_T + c, 16)
            idx_v[pl.ds(j * 16, 16)] = idx
        pltpu.async_copy(emb_hbm.at[idx_v], rows_v, sem).wait()
        pltpu.sync_copy(rows_v, pe_hbm)


def _sc_gather(ar32, emb_padded):
    mesh = plsc.VectorSubcoreMesh(core_axis_name="c", subcore_axis_name="s")
    k = functools.partial(
        pl.kernel,
        out_type=jax.ShapeDtypeStruct((BT, D), jnp.float32),
        mesh=mesh,
        scratch_types=[
            pltpu.VMEM((16,), jnp.int32),
            pltpu.VMEM((BT,), jnp.int32),
            pltpu.VMEM((BT, D), jnp.float32),
            pltpu.SemaphoreType.DMA,
        ],
    )(_sc_gather_body)
    return k(ar32, emb_padded)


NT = 205
NJ = N // NT
NCHUNK = B * NJ
RING = 6
CUTS = ((0, 103), (103, 102))
SPLIT = len(CUTS)


def _tc_add_body(gate_ref, pe_ref, x_hbm, o_hbm, bufs, in_sems, out_sems):
    g = jnp.tanh(gate_ref[0])

    def _chunk(i):
        return i // NJ, (i % NJ) * NT

    class _Par:
        def __init__(self, copies):
            self.copies = copies

        def start(self):
            for c in self.copies:
                c.start()

        def wait(self):
            for c in self.copies:
                c.wait()

    def _in_copy(k, i):
        b, n0 = _chunk(i)
        return _Par([
            pltpu.make_async_copy(
                x_hbm.at[pl.ds(b, 1), pl.ds(n0 + o, l)],
                bufs[k].at[:, pl.ds(o, l)], in_sems.at[k, s])
            for s, (o, l) in enumerate(CUTS)])

    def _out_copy(k, i):
        b, n0 = _chunk(i)
        return _Par([
            pltpu.make_async_copy(
                bufs[k].at[:, pl.ds(o, l)],
                o_hbm.at[pl.ds(b, 1), pl.ds(n0 + o, l)], out_sems.at[k, s])
            for s, (o, l) in enumerate(CUTS)])

    for k in range(RING):
        _in_copy(k, k).start()

    lookahead = RING - 2
    for i in range(NCHUNK):
        k = i % RING
        _in_copy(k, i).wait()
        b, _n0 = _chunk(i)
        pe_row = pe_ref[pl.ds(b, 1)]
        bufs[k][...] = bufs[k][...] + pe_row * g
        _out_copy(k, i).start()

        nxt = i + lookahead
        if i >= 2 and nxt < NCHUNK:
            _out_copy(nxt % RING, i - 2).wait()
            _in_copy(nxt % RING, nxt).start()

    for k in range(RING):
        _out_copy((NCHUNK - RING + k) % RING, NCHUNK - RING + k).wait()


def _tc_add(gate, pe4, xt):
    return pl.pallas_call(
        _tc_add_body,
        in_specs=[
            pl.BlockSpec(memory_space=pltpu.SMEM),
            pl.BlockSpec(memory_space=pltpu.VMEM),
            pl.BlockSpec(memory_space=pl.ANY),
        ],
        out_specs=pl.BlockSpec(memory_space=pl.ANY),
        out_shape=jax.ShapeDtypeStruct((B, N, MAX_T, D), jnp.float32),
        scratch_shapes=(
            [pltpu.VMEM((1, NT, MAX_T, D), jnp.float32)
             for _ in range(RING)],
            pltpu.SemaphoreType.DMA((RING, SPLIT)),
            pltpu.SemaphoreType.DMA((RING, SPLIT)),
        ),
    )(gate, pe4, xt)


def kernel(x, aspect_ratio, embedding, gate):
    ar32 = aspect_ratio.astype(jnp.int32).reshape(16)
    emb_flat = embedding.reshape(MAX_T * MAX_T, D)
    emb_padded = jnp.concatenate(
        [emb_flat, jnp.zeros((1, D), dtype=emb_flat.dtype)], axis=0
    )
    pe = _sc_gather(ar32, emb_padded)
    pe4 = pe.reshape(B, 1, MAX_T, D)
    xt = x.transpose(0, 2, 1, 3)
    out_t = _tc_add(gate, pe4, xt)
    return out_t.transpose(0, 2, 1, 3)


# speedup vs baseline: 1.0300x; 1.0300x over previous
import functools
import math

import jax
import jax.numpy as jnp
from jax import lax
from jax.experimental import pallas as pl
from jax.experimental.pallas import tpu as pltpu
from jax.experimental.pallas import tpu_sc as plsc

MAX_T = 4
D = 1280
B = 8
N = 1025
BT = B * MAX_T


def _vgather(vec, idx):
    return lax.gather(
        vec, idx[:, None],
        dimension_numbers=lax.GatherDimensionNumbers(
            offset_dims=(), collapsed_slice_dims=(0,), start_index_map=(0,)),
        slice_sizes=(1,),
        mode=lax.GatherScatterMode.PROMISE_IN_BOUNDS)


def _sc_gather_body(ar_hbm, emb_hbm, pe_hbm, ar_v, idx0_v, idx1_v, rows_v,
                    sem0, sem1):
    cid = lax.axis_index("c")
    sid = lax.axis_index("s")

    @pl.when(jnp.logical_and(cid == 0, sid == 0))
    def _():
        pltpu.sync_copy(ar_hbm, ar_v)
        ar = ar_v[...]
        copies = []
        for j, (idx_v, sem) in enumerate(((idx0_v, sem0), (idx1_v, sem1))):
            lane = lax.broadcasted_iota(jnp.int32, (16,), 0)
            wid = lane + j * 16
            b = lax.div(wid, 4)
            t = wid - b * 4
            h = _vgather(ar, b)
            w = _vgather(ar, 8 + b)
            ws = jnp.maximum(w, 1)
            r = lax.div(t, ws)
            c = t - r * ws
            valid = t < h * w
            idx = jnp.where(valid, r * MAX_T + c, 16)
            idx_v[...] = idx
            copies.append(pltpu.async_copy(
                emb_hbm.at[idx_v], rows_v.at[pl.ds(j * 16, 16)], sem))
        for c_ in copies:
            c_.wait()
        pltpu.sync_copy(rows_v, pe_hbm)


def _sc_gather(ar32, emb_padded):
    mesh = plsc.VectorSubcoreMesh(core_axis_name="c", subcore_axis_name="s")
    k = functools.partial(
        pl.kernel,
        out_type=jax.ShapeDtypeStruct((BT, D), jnp.float32),
        mesh=mesh,
        scratch_types=[
            pltpu.VMEM((16,), jnp.int32),
            pltpu.VMEM((16,), jnp.int32),
            pltpu.VMEM((16,), jnp.int32),
            pltpu.VMEM((BT, D), jnp.float32),
            pltpu.SemaphoreType.DMA,
            pltpu.SemaphoreType.DMA,
        ],
    )(_sc_gather_body)
    return k(ar32, emb_padded)


NT = 205
NJ = N // NT
NCHUNK = B * NJ
RING = 6
CUTS = ((0, 103), (103, 102))
SPLIT = len(CUTS)


def _tc_add_body(gate_ref, pe_hbm, x_hbm, o_hbm, bufs, pe_v,
                 in_sems, out_sems, pe_sem):
    pe_cp = pltpu.make_async_copy(pe_hbm, pe_v, pe_sem)
    pe_cp.start()
    g = jnp.tanh(gate_ref[0])

    def _chunk(i):
        return i // NJ, (i % NJ) * NT

    class _Par:
        def __init__(self, copies):
            self.copies = copies

        def start(self):
            for c in self.copies:
                c.start()

        def wait(self):
            for c in self.copies:
                c.wait()

    def _in_copy(k, i):
        b, n0 = _chunk(i)
        return _Par([
            pltpu.make_async_copy(
                x_hbm.at[pl.ds(b, 1), pl.ds(n0 + o, l)],
                bufs[k].at[:, pl.ds(o, l)], in_sems.at[k, s])
            for s, (o, l) in enumerate(CUTS)])

    def _out_copy(k, i):
        b, n0 = _chunk(i)
        return _Par([
            pltpu.make_async_copy(
                bufs[k].at[:, pl.ds(o, l)],
                o_hbm.at[pl.ds(b, 1), pl.ds(n0 + o, l)], out_sems.at[k, s])
            for s, (o, l) in enumerate(CUTS)])

    for k in range(RING):
        _in_copy(k, k).start()
    pe_cp.wait()

    lookahead = RING - 2
    for i in range(NCHUNK):
        k = i % RING
        _in_copy(k, i).wait()
        b, _n0 = _chunk(i)
        pe_rows = pe_v[pl.ds(MAX_T * b, MAX_T), :]
        bufs[k][...] = bufs[k][...] + pe_rows[None, None] * g
        _out_copy(k, i).start()

        nxt = i + lookahead
        if i >= 2 and nxt < NCHUNK:
            _out_copy(nxt % RING, i - 2).wait()
            _in_copy(nxt % RING, nxt).start()

    for k in range(RING):
        _out_copy((NCHUNK - RING + k) % RING, NCHUNK - RING + k).wait()


def _tc_add(gate, pe, xt):
    return pl.pallas_call(
        _tc_add_body,
        in_specs=[
            pl.BlockSpec(memory_space=pltpu.SMEM),
            pl.BlockSpec(memory_space=pl.ANY),
            pl.BlockSpec(memory_space=pl.ANY),
        ],
        out_specs=pl.BlockSpec(memory_space=pl.ANY),
        out_shape=jax.ShapeDtypeStruct((B, N, MAX_T, D), jnp.float32),
        scratch_shapes=(
            [pltpu.VMEM((1, NT, MAX_T, D), jnp.float32)
             for _ in range(RING)],
            pltpu.VMEM((BT, D), jnp.float32),
            pltpu.SemaphoreType.DMA((RING, SPLIT)),
            pltpu.SemaphoreType.DMA((RING, SPLIT)),
            pltpu.SemaphoreType.DMA,
        ),
    )(gate, pe, xt)


def kernel(x, aspect_ratio, embedding, gate):
    ar32 = aspect_ratio.astype(jnp.int32).T.reshape(16)
    emb_flat = embedding.reshape(MAX_T * MAX_T, D)
    emb_padded = jnp.concatenate(
        [emb_flat, jnp.zeros((1, D), dtype=emb_flat.dtype)], axis=0
    )
    pe = _sc_gather(ar32, emb_padded)
    xt = x.transpose(0, 2, 1, 3)
    out_t = _tc_add(gate, pe, xt)
    return out_t.transpose(0, 2, 1, 3)


# speedup vs baseline: 1.0566x; 1.0258x over previous
import functools
import math

import jax
import jax.numpy as jnp
from jax import lax
from jax.experimental import pallas as pl
from jax.experimental.pallas import tpu as pltpu
from jax.experimental.pallas import tpu_sc as plsc

MAX_T = 4
D = 1280
B = 8
N = 1025
BT = B * MAX_T


def _vgather(vec, idx):
    return lax.gather(
        vec, idx[:, None],
        dimension_numbers=lax.GatherDimensionNumbers(
            offset_dims=(), collapsed_slice_dims=(0,), start_index_map=(0,)),
        slice_sizes=(1,),
        mode=lax.GatherScatterMode.PROMISE_IN_BOUNDS)


def _sc_gather_body(ar_hbm, emb_hbm, pe_hbm, ar_v, idx0_v, idx1_v, rows_v,
                    sem0, sem1):
    cid = lax.axis_index("c")
    sid = lax.axis_index("s")

    @pl.when(jnp.logical_and(cid == 0, sid == 0))
    def _():
        pltpu.sync_copy(ar_hbm, ar_v)
        ar = ar_v[...]
        copies = []
        for j, (idx_v, sem) in enumerate(((idx0_v, sem0), (idx1_v, sem1))):
            lane = lax.broadcasted_iota(jnp.int32, (16,), 0)
            wid = lane + j * 16
            b = lax.div(wid, 4)
            t = wid - b * 4
            h = _vgather(ar, b)
            w = _vgather(ar, 8 + b)
            ws = jnp.maximum(w, 1)
            r = lax.div(t, ws)
            c = t - r * ws
            valid = t < h * w
            idx = jnp.where(valid, r * MAX_T + c, 16)
            idx_v[...] = idx
            copies.append(pltpu.async_copy(
                emb_hbm.at[idx_v], rows_v.at[pl.ds(j * 16, 16)], sem))
        for c_ in copies:
            c_.wait()
        pltpu.sync_copy(rows_v, pe_hbm)


def _scs_gather_body(ar_hbm, emb_hbm, pe_hbm, ar_s, sem):
    cid = lax.axis_index("c")

    @pl.when(cid == 0)
    def _():
        pltpu.sync_copy(ar_hbm, ar_s)
        copies = []
        for wid in range(BT):
            b, t = wid // MAX_T, wid % MAX_T
            h = ar_s[b]
            w = ar_s[B + b]
            ws = jnp.maximum(w, 1)
            r = lax.div(t, ws)
            c = t - r * ws
            valid = t < h * w
            idx = jnp.where(valid, r * MAX_T + c, 16)
            copies.append(pltpu.async_copy(
                emb_hbm.at[pl.ds(idx, 1)], pe_hbm.at[pl.ds(wid, 1)], sem))
        for c_ in copies:
            c_.start()
        for c_ in copies:
            c_.wait()


def _sc_gather(ar32, emb_padded):
    mesh = plsc.ScalarSubcoreMesh(axis_name="c", num_cores=2)
    k = functools.partial(
        pl.kernel,
        out_type=jax.ShapeDtypeStruct((BT, D), jnp.float32),
        mesh=mesh,
        scratch_types=[
            pltpu.SMEM((16,), jnp.int32),
            pltpu.SemaphoreType.DMA,
        ],
    )(_scs_gather_body)
    return k(ar32, emb_padded)


NT = 205
NJ = N // NT
NCHUNK = B * NJ
RING = 6
CUTS = ((0, 103), (103, 102))
SPLIT = len(CUTS)


def _tc_add_body(gate_ref, pe_hbm, x_hbm, o_hbm, bufs, pe_v,
                 in_sems, out_sems, pe_sem):
    pe_cp = pltpu.make_async_copy(pe_hbm, pe_v, pe_sem)
    pe_cp.start()
    g = jnp.tanh(gate_ref[0])

    def _chunk(i):
        return i // NJ, (i % NJ) * NT

    class _Par:
        def __init__(self, copies):
            self.copies = copies

        def start(self):
            for c in self.copies:
                c.start()

        def wait(self):
            for c in self.copies:
                c.wait()

    def _in_copy(k, i):
        b, n0 = _chunk(i)
        return _Par([
            pltpu.make_async_copy(
                x_hbm.at[pl.ds(b, 1), pl.ds(n0 + o, l)],
                bufs[k].at[:, pl.ds(o, l)], in_sems.at[k, s])
            for s, (o, l) in enumerate(CUTS)])

    def _out_copy(k, i):
        b, n0 = _chunk(i)
        return _Par([
            pltpu.make_async_copy(
                bufs[k].at[:, pl.ds(o, l)],
                o_hbm.at[pl.ds(b, 1), pl.ds(n0 + o, l)], out_sems.at[k, s])
            for s, (o, l) in enumerate(CUTS)])

    for k in range(RING):
        _in_copy(k, k).start()
    pe_cp.wait()

    lookahead = RING - 2
    for i in range(NCHUNK):
        k = i % RING
        _in_copy(k, i).wait()
        b, _n0 = _chunk(i)
        pe_rows = pe_v[pl.ds(MAX_T * b, MAX_T), :]
        bufs[k][...] = bufs[k][...] + pe_rows[None, None] * g
        _out_copy(k, i).start()

        nxt = i + lookahead
        if i >= 2 and nxt < NCHUNK:
            _out_copy(nxt % RING, i - 2).wait()
            _in_copy(nxt % RING, nxt).start()

    for k in range(RING):
        _out_copy((NCHUNK - RING + k) % RING, NCHUNK - RING + k).wait()


def _tc_add(gate, pe, xt):
    return pl.pallas_call(
        _tc_add_body,
        in_specs=[
            pl.BlockSpec(memory_space=pltpu.SMEM),
            pl.BlockSpec(memory_space=pl.ANY),
            pl.BlockSpec(memory_space=pl.ANY),
        ],
        out_specs=pl.BlockSpec(memory_space=pl.ANY),
        out_shape=jax.ShapeDtypeStruct((B, N, MAX_T, D), jnp.float32),
        scratch_shapes=(
            [pltpu.VMEM((1, NT, MAX_T, D), jnp.float32)
             for _ in range(RING)],
            pltpu.VMEM((BT, D), jnp.float32),
            pltpu.SemaphoreType.DMA((RING, SPLIT)),
            pltpu.SemaphoreType.DMA((RING, SPLIT)),
            pltpu.SemaphoreType.DMA,
        ),
    )(gate, pe, xt)


def kernel(x, aspect_ratio, embedding, gate):
    ar32 = aspect_ratio.astype(jnp.int32).T.reshape(16)
    emb_flat = embedding.reshape(MAX_T * MAX_T, D)
    emb_padded = jnp.concatenate(
        [emb_flat, jnp.zeros((1, D), dtype=emb_flat.dtype)], axis=0
    )
    pe = _sc_gather(ar32, emb_padded)
    xt = x.transpose(0, 2, 1, 3)
    out_t = _tc_add(gate, pe, xt)
    return out_t.transpose(0, 2, 1, 3)


# speedup vs baseline: 1.0749x; 1.0173x over previous
import functools
import math

import jax
import jax.numpy as jnp
from jax import lax
from jax.experimental import pallas as pl
from jax.experimental.pallas import tpu as pltpu
from jax.experimental.pallas import tpu_sc as plsc

MAX_T = 4
D = 1280
B = 8
N = 1025
BT = B * MAX_T


def _vgather(vec, idx):
    return lax.gather(
        vec, idx[:, None],
        dimension_numbers=lax.GatherDimensionNumbers(
            offset_dims=(), collapsed_slice_dims=(0,), start_index_map=(0,)),
        slice_sizes=(1,),
        mode=lax.GatherScatterMode.PROMISE_IN_BOUNDS)


def _sc_gather_body(ar_hbm, emb_hbm, pe_hbm, ar_v, idx0_v, idx1_v, rows_v,
                    sem0, sem1):
    cid = lax.axis_index("c")
    sid = lax.axis_index("s")

    @pl.when(jnp.logical_and(cid == 0, sid == 0))
    def _():
        pltpu.sync_copy(ar_hbm, ar_v)
        ar = ar_v[...]
        copies = []
        for j, (idx_v, sem) in enumerate(((idx0_v, sem0), (idx1_v, sem1))):
            lane = lax.broadcasted_iota(jnp.int32, (16,), 0)
            wid = lane + j * 16
            b = lax.div(wid, 4)
            t = wid - b * 4
            h = _vgather(ar, b)
            w = _vgather(ar, 8 + b)
            ws = jnp.maximum(w, 1)
            r = lax.div(t, ws)
            c = t - r * ws
            valid = t < h * w
            idx = jnp.where(valid, r * MAX_T + c, 16)
            idx_v[...] = idx
            copies.append(pltpu.async_copy(
                emb_hbm.at[idx_v], rows_v.at[pl.ds(j * 16, 16)], sem))
        for c_ in copies:
            c_.wait()
        pltpu.sync_copy(rows_v, pe_hbm)


def _scs_gather_body(ar_hbm, emb_hbm, pe_hbm, ar_s, sem):
    cid = lax.axis_index("c")

    @pl.when(cid == 0)
    def _():
        pltpu.sync_copy(ar_hbm, ar_s)
        copies = []
        for wid in range(BT):
            b, t = wid // MAX_T, wid % MAX_T
            w = ar_s[1, b]
            ws = jnp.maximum(w, 1)
            r = lax.div(t, ws)
            c = t - r * ws
            copies.append(pltpu.async_copy(
                emb_hbm.at[r, c, 0], pe_hbm.at[wid], sem))
        for c_ in copies:
            c_.start()
        for c_ in copies:
            c_.wait()


def _sc_gather(ar2, embedding):
    mesh = plsc.ScalarSubcoreMesh(axis_name="c", num_cores=2)
    k = functools.partial(
        pl.kernel,
        out_type=jax.ShapeDtypeStruct((BT, D), jnp.float32),
        mesh=mesh,
        scratch_types=[
            pltpu.SMEM((2, B), jnp.int32),
            pltpu.SemaphoreType.DMA,
        ],
    )(_scs_gather_body)
    return k(ar2, embedding)


NT = 205
NJ = N // NT
NCHUNK = B * NJ
RING = 6
CUTS = ((0, 103), (103, 102))
SPLIT = len(CUTS)


def _tc_add_body(gate_ref, ar_ref, pe_hbm, x_hbm, o_hbm, bufs, pe_v,
                 in_sems, out_sems, pe_sem):
    pe_cp = pltpu.make_async_copy(pe_hbm, pe_v, pe_sem)
    pe_cp.start()
    g = jnp.tanh(gate_ref[0])

    def _chunk(i):
        return i // NJ, (i % NJ) * NT

    class _Par:
        def __init__(self, copies):
            self.copies = copies

        def start(self):
            for c in self.copies:
                c.start()

        def wait(self):
            for c in self.copies:
                c.wait()

    def _in_copy(k, i):
        b, n0 = _chunk(i)
        return _Par([
            pltpu.make_async_copy(
                x_hbm.at[pl.ds(b, 1), pl.ds(n0 + o, l)],
                bufs[k].at[:, pl.ds(o, l)], in_sems.at[k, s])
            for s, (o, l) in enumerate(CUTS)])

    def _out_copy(k, i):
        b, n0 = _chunk(i)
        return _Par([
            pltpu.make_async_copy(
                bufs[k].at[:, pl.ds(o, l)],
                o_hbm.at[pl.ds(b, 1), pl.ds(n0 + o, l)], out_sems.at[k, s])
            for s, (o, l) in enumerate(CUTS)])

    for k in range(RING):
        _in_copy(k, k).start()
    pe_cp.wait()

    lookahead = RING - 2
    for i in range(NCHUNK):
        k = i % RING
        _in_copy(k, i).wait()
        b, _n0 = _chunk(i)
        pe_rows = pe_v[pl.ds(MAX_T * b, MAX_T), :]
        hw = ar_ref[0, b] * ar_ref[1, b]
        tvec = lax.broadcasted_iota(jnp.int32, (MAX_T, 1), 0)
        scale = jnp.where(tvec < hw, g, 0.0)
        bufs[k][...] = bufs[k][...] + pe_rows[None, None] * scale[None, None]
        _out_copy(k, i).start()

        nxt = i + lookahead
        if i >= 2 and nxt < NCHUNK:
            _out_copy(nxt % RING, i - 2).wait()
            _in_copy(nxt % RING, nxt).start()

    for k in range(RING):
        _out_copy((NCHUNK - RING + k) % RING, NCHUNK - RING + k).wait()


def _tc_add(gate, ar2, pe, xt):
    return pl.pallas_call(
        _tc_add_body,
        in_specs=[
            pl.BlockSpec(memory_space=pltpu.SMEM),
            pl.BlockSpec(memory_space=pltpu.SMEM),
            pl.BlockSpec(memory_space=pl.ANY),
            pl.BlockSpec(memory_space=pl.ANY),
        ],
        out_specs=pl.BlockSpec(memory_space=pl.ANY),
        out_shape=jax.ShapeDtypeStruct((B, N, MAX_T, D), jnp.float32),
        scratch_shapes=(
            [pltpu.VMEM((1, NT, MAX_T, D), jnp.float32)
             for _ in range(RING)],
            pltpu.VMEM((BT, D), jnp.float32),
            pltpu.SemaphoreType.DMA((RING, SPLIT)),
            pltpu.SemaphoreType.DMA((RING, SPLIT)),
            pltpu.SemaphoreType.DMA,
        ),
    )(gate, ar2, pe, xt)


def kernel(x, aspect_ratio, embedding, gate):
    ar2 = aspect_ratio.astype(jnp.int32).T
    pe = _sc_gather(ar2, embedding)
    xt = x.transpose(0, 2, 1, 3)
    out_t = _tc_add(gate, ar2, pe, xt)
    return out_t.transpose(0, 2, 1, 3)


# speedup vs baseline: 1.0836x; 1.0081x over previous
import functools
import math

import jax
import jax.numpy as jnp
from jax import lax
from jax.experimental import pallas as pl
from jax.experimental.pallas import tpu as pltpu
from jax.experimental.pallas import tpu_sc as plsc

MAX_T = 4
D = 1280
B = 8
N = 1025
BT = B * MAX_T


def _vgather(vec, idx):
    return lax.gather(
        vec, idx[:, None],
        dimension_numbers=lax.GatherDimensionNumbers(
            offset_dims=(), collapsed_slice_dims=(0,), start_index_map=(0,)),
        slice_sizes=(1,),
        mode=lax.GatherScatterMode.PROMISE_IN_BOUNDS)


def _sc_gather_body(ar_hbm, emb_hbm, pe_hbm, ar_v, idx0_v, idx1_v, rows_v,
                    sem0, sem1):
    cid = lax.axis_index("c")
    sid = lax.axis_index("s")

    @pl.when(jnp.logical_and(cid == 0, sid == 0))
    def _():
        pltpu.sync_copy(ar_hbm, ar_v)
        ar = ar_v[...]
        copies = []
        for j, (idx_v, sem) in enumerate(((idx0_v, sem0), (idx1_v, sem1))):
            lane = lax.broadcasted_iota(jnp.int32, (16,), 0)
            wid = lane + j * 16
            b = lax.div(wid, 4)
            t = wid - b * 4
            h = _vgather(ar, b)
            w = _vgather(ar, 8 + b)
            ws = jnp.maximum(w, 1)
            r = lax.div(t, ws)
            c = t - r * ws
            valid = t < h * w
            idx = jnp.where(valid, r * MAX_T + c, 16)
            idx_v[...] = idx
            copies.append(pltpu.async_copy(
                emb_hbm.at[idx_v], rows_v.at[pl.ds(j * 16, 16)], sem))
        for c_ in copies:
            c_.wait()
        pltpu.sync_copy(rows_v, pe_hbm)


def _scs_gather_body(ar_hbm, emb_hbm, pe_hbm, ar_s, sem):
    cid = lax.axis_index("c")

    @pl.when(cid == 0)
    def _():
        pltpu.sync_copy(ar_hbm, ar_s)
        copies = []
        for wid in range(BT):
            b, t = wid // MAX_T, wid % MAX_T
            w = ar_s[b, 1]
            ws = jnp.maximum(w, 1)
            r = lax.div(t, ws)
            c = t - r * ws
            copies.append(pltpu.async_copy(
                emb_hbm.at[r, c, 0], pe_hbm.at[wid], sem))
        for c_ in copies:
            c_.start()
        for c_ in copies:
            c_.wait()


def _sc_gather(ar2, embedding):
    mesh = plsc.ScalarSubcoreMesh(axis_name="c", num_cores=1)
    k = functools.partial(
        pl.kernel,
        out_type=jax.ShapeDtypeStruct((BT, D), jnp.float32),
        mesh=mesh,
        scratch_types=[
            pltpu.SMEM((B, 2), jnp.int32),
            pltpu.SemaphoreType.DMA,
        ],
    )(_scs_gather_body)
    return k(ar2, embedding)


NT = 205
NJ = N // NT
NCHUNK = B * NJ
RING = 6
CUTS = ((0, 103), (103, 102))
SPLIT = len(CUTS)


def _tc_add_body(gate_ref, ar_ref, pe_hbm, x_hbm, o_hbm, bufs, pe_v,
                 in_sems, out_sems, pe_sem):
    pe_cp = pltpu.make_async_copy(pe_hbm, pe_v, pe_sem)
    pe_cp.start()
    g = jnp.tanh(gate_ref[0])

    def _chunk(i):
        return i // NJ, (i % NJ) * NT

    class _Par:
        def __init__(self, copies):
            self.copies = copies

        def start(self):
            for c in self.copies:
                c.start()

        def wait(self):
            for c in self.copies:
                c.wait()

    def _in_copy(k, i):
        b, n0 = _chunk(i)
        return _Par([
            pltpu.make_async_copy(
                x_hbm.at[pl.ds(b, 1), pl.ds(n0 + o, l)],
                bufs[k].at[:, pl.ds(o, l)], in_sems.at[k, s])
            for s, (o, l) in enumerate(CUTS)])

    def _out_copy(k, i):
        b, n0 = _chunk(i)
        return _Par([
            pltpu.make_async_copy(
                bufs[k].at[:, pl.ds(o, l)],
                o_hbm.at[pl.ds(b, 1), pl.ds(n0 + o, l)], out_sems.at[k, s])
            for s, (o, l) in enumerate(CUTS)])

    for k in range(RING):
        _in_copy(k, k).start()
    pe_cp.wait()

    lookahead = RING - 2
    for i in range(NCHUNK):
        k = i % RING
        _in_copy(k, i).wait()
        b, _n0 = _chunk(i)
        pe_rows = pe_v[pl.ds(MAX_T * b, MAX_T), :]
        hw = ar_ref[b, 0] * ar_ref[b, 1]
        tvec = lax.broadcasted_iota(jnp.int32, (MAX_T, 1), 0)
        scale = jnp.where(tvec < hw, g, 0.0)
        bufs[k][...] = bufs[k][...] + pe_rows[None, None] * scale[None, None]
        _out_copy(k, i).start()

        nxt = i + lookahead
        if i >= 2 and nxt < NCHUNK:
            _out_copy(nxt % RING, i - 2).wait()
            _in_copy(nxt % RING, nxt).start()

    for k in range(RING):
        _out_copy((NCHUNK - RING + k) % RING, NCHUNK - RING + k).wait()


def _tc_add(gate, ar2, pe, xt):
    return pl.pallas_call(
        _tc_add_body,
        in_specs=[
            pl.BlockSpec(memory_space=pltpu.SMEM),
            pl.BlockSpec(memory_space=pltpu.SMEM),
            pl.BlockSpec(memory_space=pl.ANY),
            pl.BlockSpec(memory_space=pl.ANY),
        ],
        out_specs=pl.BlockSpec(memory_space=pl.ANY),
        out_shape=jax.ShapeDtypeStruct((B, N, MAX_T, D), jnp.float32),
        scratch_shapes=(
            [pltpu.VMEM((1, NT, MAX_T, D), jnp.float32)
             for _ in range(RING)],
            pltpu.VMEM((BT, D), jnp.float32),
            pltpu.SemaphoreType.DMA((RING, SPLIT)),
            pltpu.SemaphoreType.DMA((RING, SPLIT)),
            pltpu.SemaphoreType.DMA,
        ),
    )(gate, ar2, pe, xt)


def kernel(x, aspect_ratio, embedding, gate):
    pe = _sc_gather(aspect_ratio, embedding)
    xt = x.transpose(0, 2, 1, 3)
    out_t = _tc_add(gate, aspect_ratio, pe, xt)
    return out_t.transpose(0, 2, 1, 3)


# speedup vs baseline: 1.0840x; 1.0004x over previous
import functools

import jax
import jax.numpy as jnp
from jax import lax
from jax.experimental import pallas as pl
from jax.experimental.pallas import tpu as pltpu
from jax.experimental.pallas import tpu_sc as plsc

MAX_T = 4
D = 1280
B = 8
N = 1025
BT = B * MAX_T


def _scs_gather_body(ar_hbm, emb_hbm, pe_hbm, ar_s, sem):
    cid = lax.axis_index("c")

    @pl.when(cid == 0)
    def _():
        pltpu.sync_copy(ar_hbm, ar_s)
        copies = []
        for wid in range(BT):
            b, t = wid // MAX_T, wid % MAX_T
            w = ar_s[b, 1]
            ws = jnp.maximum(w, 1)
            r = lax.div(t, ws)
            c = t - r * ws
            copies.append(pltpu.async_copy(
                emb_hbm.at[r, c, 0], pe_hbm.at[wid], sem))
        for c_ in copies:
            c_.start()
        for c_ in copies:
            c_.wait()


def _sc_gather(ar2, embedding):
    mesh = plsc.ScalarSubcoreMesh(axis_name="c", num_cores=1)
    k = functools.partial(
        pl.kernel,
        out_type=jax.ShapeDtypeStruct((BT, D), jnp.float32),
        mesh=mesh,
        scratch_types=[
            pltpu.SMEM((B, 2), jnp.int32),
            pltpu.SemaphoreType.DMA,
        ],
    )(_scs_gather_body)
    return k(ar2, embedding)


NT = 205
NJ = N // NT
NCHUNK = B * NJ
RING = 6
CUTS = ((0, 103), (103, 102))
SPLIT = len(CUTS)


def _tc_add_body(gate_ref, ar_ref, pe_hbm, x_hbm, o_hbm, bufs, pe_v,
                 in_sems, out_sems, pe_sem):
    pe_cp = pltpu.make_async_copy(pe_hbm, pe_v, pe_sem)
    pe_cp.start()
    g = jnp.tanh(gate_ref[0])

    def _chunk(i):
        return i // NJ, (i % NJ) * NT

    class _Par:
        def __init__(self, copies):
            self.copies = copies

        def start(self):
            for c in self.copies:
                c.start()

        def wait(self):
            for c in self.copies:
                c.wait()

    def _in_copy(k, i):
        b, n0 = _chunk(i)
        return _Par([
            pltpu.make_async_copy(
                x_hbm.at[pl.ds(b, 1), pl.ds(n0 + o, l)],
                bufs[k].at[:, pl.ds(o, l)], in_sems.at[k, s])
            for s, (o, l) in enumerate(CUTS)])

    def _out_copy(k, i):
        b, n0 = _chunk(i)
        return _Par([
            pltpu.make_async_copy(
                bufs[k].at[:, pl.ds(o, l)],
                o_hbm.at[pl.ds(b, 1), pl.ds(n0 + o, l)], out_sems.at[k, s])
            for s, (o, l) in enumerate(CUTS)])

    for k in range(RING):
        _in_copy(k, k).start()
    pe_cp.wait()

    lookahead = RING - 2
    for i in range(NCHUNK):
        k = i % RING
        _in_copy(k, i).wait()
        b, _n0 = _chunk(i)
        pe_rows = pe_v[pl.ds(MAX_T * b, MAX_T), :]
        hw = ar_ref[b, 0] * ar_ref[b, 1]
        tvec = lax.broadcasted_iota(jnp.int32, (MAX_T, 1), 0)
        scale = jnp.where(tvec < hw, g, 0.0)
        bufs[k][...] = bufs[k][...] + pe_rows[None, None] * scale[None, None]
        _out_copy(k, i).start()

        nxt = i + lookahead
        if i >= 2 and nxt < NCHUNK:
            _out_copy(nxt % RING, i - 2).wait()
            _in_copy(nxt % RING, nxt).start()

    for k in range(RING):
        _out_copy((NCHUNK - RING + k) % RING, NCHUNK - RING + k).wait()


def _tc_add(gate, ar2, pe, xt):
    return pl.pallas_call(
        _tc_add_body,
        in_specs=[
            pl.BlockSpec(memory_space=pltpu.SMEM),
            pl.BlockSpec(memory_space=pltpu.SMEM),
            pl.BlockSpec(memory_space=pl.ANY),
            pl.BlockSpec(memory_space=pl.ANY),
        ],
        out_specs=pl.BlockSpec(memory_space=pl.ANY),
        out_shape=jax.ShapeDtypeStruct((B, N, MAX_T, D), jnp.float32),
        scratch_shapes=(
            [pltpu.VMEM((1, NT, MAX_T, D), jnp.float32)
             for _ in range(RING)],
            pltpu.VMEM((BT, D), jnp.float32),
            pltpu.SemaphoreType.DMA((RING, SPLIT)),
            pltpu.SemaphoreType.DMA((RING, SPLIT)),
            pltpu.SemaphoreType.DMA,
        ),
    )(gate, ar2, pe, xt)


def kernel(x, aspect_ratio, embedding, gate):
    pe = _sc_gather(aspect_ratio, embedding)
    xt = x.transpose(0, 2, 1, 3)
    out_t = _tc_add(gate, aspect_ratio, pe, xt)
    return out_t.transpose(0, 2, 1, 3)
